# baseline (speedup 1.0000x reference)
import functools

import jax
import jax.numpy as jnp
from jax import lax
from jax.experimental import pallas as pl
from jax.experimental.pallas import tpu as pltpu
from jax.experimental.pallas import tpu_sc as plsc

N = 10000
NPAD = 10240
E = 320000
CH = 128
NCPT = 80
EPT = NCPT * CH
EPAD = 32 * EPT
NBUF = 4
D_IN = 128
D_OUT = 64
ROWS_PT = NPAD // 16

_mesh = plsc.VectorSubcoreMesh(core_axis_name="c", subcore_axis_name="s")


@functools.partial(
    pl.kernel,
    out_type=jax.ShapeDtypeStruct((2, NPAD), jnp.float32),
    mesh=_mesh,
    compiler_params=pltpu.CompilerParams(use_tc_tiling_on_sc=False),
    scratch_types=[
        pltpu.VMEM((NCPT, CH), jnp.int32),
        pltpu.VMEM((CH,), jnp.float32),
        pltpu.VMEM_SHARED((NPAD,), jnp.float32),
        pltpu.SemaphoreType.DMA,
    ],
)
def _deg_kernel(ei_hbm, zeros_hbm, out_hbm, idx_d, ones_v, acc, sem):
    cid = lax.axis_index("c")
    sid = lax.axis_index("s")
    wid = cid * 16 + sid
    for i in range(CH // 16):
        ones_v[pl.ds(i * 16, 16)] = jnp.full((16,), 1.0, jnp.float32)
    pltpu.sync_copy(ei_hbm.at[1, wid], idx_d)
    pltpu.sync_copy(zeros_hbm.at[pl.ds(sid * ROWS_PT, ROWS_PT)],
                    acc.at[pl.ds(sid * ROWS_PT, ROWS_PT)])
    plsc.subcore_barrier()

    def fire(c, carry):
        pltpu.async_copy(ones_v, acc.at[idx_d.at[c]], sem, add=True)
        return carry

    def drain(c, carry):
        pltpu.make_async_copy(zeros_hbm.at[pl.ds(0, CH)], ones_v, sem).wait()
        return carry

    def group(t, carry):
        lax.fori_loop(16 * t, 16 * t + 16, fire, 0)
        lax.fori_loop(0, 16, drain, 0)
        return carry

    lax.fori_loop(0, NCPT // 16, group, 0)
    plsc.subcore_barrier()
    pltpu.sync_copy(acc.at[pl.ds(sid * ROWS_PT, ROWS_PT)],
                    out_hbm.at[cid, pl.ds(sid * ROWS_PT, ROWS_PT)])


@functools.partial(
    pl.kernel,
    out_type=jax.ShapeDtypeStruct((2, NPAD, D_OUT), jnp.float32),
    mesh=_mesh,
    compiler_params=pltpu.CompilerParams(use_tc_tiling_on_sc=False),
    scratch_types=[
        pltpu.VMEM((NCPT, CH), jnp.int32),
        pltpu.VMEM((NCPT, CH), jnp.int32),
        pltpu.VMEM((NBUF, CH, D_OUT), jnp.float32),
        pltpu.VMEM_SHARED((NPAD, D_OUT), jnp.float32),
        [pltpu.SemaphoreType.DMA] * NBUF,
        [pltpu.SemaphoreType.DMA] * NBUF,
    ],
)
def _agg_kernel(hs_hbm, ei_hbm, out_hbm, idx_s, idx_d, rows, acc, gsem, ssem):
    cid = lax.axis_index("c")
    sid = lax.axis_index("s")
    wid = cid * 16 + sid
    pltpu.sync_copy(ei_hbm.at[0, wid], idx_s)
    pltpu.sync_copy(ei_hbm.at[1, wid], idx_d)
    pltpu.sync_copy(hs_hbm.at[pl.ds(sid * ROWS_PT, ROWS_PT)],
                    acc.at[pl.ds(sid * ROWS_PT, ROWS_PT)])
    plsc.subcore_barrier()

    def group(t, carry):
        for b in range(NBUF):
            c = NBUF * t + b

            @pl.when(t > 0)
            def _():
                pltpu.make_async_copy(hs_hbm.at[pl.ds(0, CH)], rows.at[b],
                                      ssem[b]).wait()

            pltpu.async_copy(hs_hbm.at[idx_s.at[c]], rows.at[b], gsem[b])
        for b in range(NBUF):
            c = NBUF * t + b
            pltpu.make_async_copy(hs_hbm.at[pl.ds(0, CH)], rows.at[b],
                                  gsem[b]).wait()
            pltpu.async_copy(rows.at[b], acc.at[idx_d.at[c]], ssem[b],
                             add=True)
        return carry

    lax.fori_loop(0, NCPT // NBUF, group, 0)
    for b in range(NBUF):
        pltpu.make_async_copy(hs_hbm.at[pl.ds(0, CH)], rows.at[b],
                              ssem[b]).wait()
    plsc.subcore_barrier()
    pltpu.sync_copy(acc.at[pl.ds(sid * ROWS_PT, ROWS_PT)],
                    out_hbm.at[cid, pl.ds(sid * ROWS_PT, ROWS_PT)])


_R = 640


def _dense1_body(x_ref, w_ref, dega_ref, degb_ref, hs_ref, dinv_ref):
    h = jnp.dot(x_ref[...], w_ref[...], preferred_element_type=jnp.float32)
    deg = dega_ref[...] + degb_ref[...] + 1.0
    dinv = lax.rsqrt(deg)
    hs_ref[...] = h * dinv
    dinv_ref[...] = dinv


def _dense1(x, W1, dega, degb):
    return pl.pallas_call(
        _dense1_body,
        grid=(NPAD // _R,),
        in_specs=[
            pl.BlockSpec((_R, D_IN), lambda i: (i, 0)),
            pl.BlockSpec((D_IN, D_OUT), lambda i: (0, 0)),
            pl.BlockSpec((_R, 1), lambda i: (i, 0)),
            pl.BlockSpec((_R, 1), lambda i: (i, 0)),
        ],
        out_specs=[
            pl.BlockSpec((_R, D_OUT), lambda i: (i, 0)),
            pl.BlockSpec((_R, 1), lambda i: (i, 0)),
        ],
        out_shape=[
            jax.ShapeDtypeStruct((NPAD, D_OUT), jnp.float32),
            jax.ShapeDtypeStruct((NPAD, 1), jnp.float32),
        ],
    )(x, W1, dega, degb)


def _dense2_body(a_ref, b_ref, hs1_ref, dinv_ref, b1_ref, g_ref, be_ref,
                 w2_ref, w3_ref, b3_ref, hs2_ref, out2_ref):
    dinv = dinv_ref[...]
    y = (a_ref[...] + b_ref[...] - hs1_ref[...]) * dinv + b1_ref[...]
    mu = jnp.mean(y, axis=-1, keepdims=True)
    d = y - mu
    var = jnp.mean(d * d, axis=-1, keepdims=True)
    hr = d * lax.rsqrt(var + 1e-5) * g_ref[...] + be_ref[...]
    hr = jnp.maximum(hr, 0.0)
    hs2_ref[...] = jnp.dot(hr, w2_ref[...],
                           preferred_element_type=jnp.float32) * dinv
    z = jnp.dot(hr, w3_ref[...], preferred_element_type=jnp.float32)
    out2_ref[...] = jax.nn.sigmoid(z + b3_ref[...])


def _dense2(a, b, hs1, dinv, b1, gamma, beta, W2, W3, b3):
    return pl.pallas_call(
        _dense2_body,
        grid=(NPAD // _R,),
        in_specs=[
            pl.BlockSpec((_R, D_OUT), lambda i: (i, 0)),
            pl.BlockSpec((_R, D_OUT), lambda i: (i, 0)),
            pl.BlockSpec((_R, D_OUT), lambda i: (i, 0)),
            pl.BlockSpec((_R, 1), lambda i: (i, 0)),
            pl.BlockSpec((1, D_OUT), lambda i: (0, 0)),
            pl.BlockSpec((1, D_OUT), lambda i: (0, 0)),
            pl.BlockSpec((1, D_OUT), lambda i: (0, 0)),
            pl.BlockSpec((D_OUT, D_OUT), lambda i: (0, 0)),
            pl.BlockSpec((D_OUT, 6), lambda i: (0, 0)),
            pl.BlockSpec((1, 6), lambda i: (0, 0)),
        ],
        out_specs=[
            pl.BlockSpec((_R, D_OUT), lambda i: (i, 0)),
            pl.BlockSpec((_R, 6), lambda i: (i, 0)),
        ],
        out_shape=[
            jax.ShapeDtypeStruct((NPAD, D_OUT), jnp.float32),
            jax.ShapeDtypeStruct((NPAD, 6), jnp.float32),
        ],
    )(a, b, hs1, dinv, b1, gamma, beta, W2, W3, b3)


def _dense3_body(a_ref, b_ref, hs2_ref, dinv_ref, b2_ref, out_ref):
    out_ref[...] = ((a_ref[...] + b_ref[...] - hs2_ref[...]) * dinv_ref[...]
                    + b2_ref[...])


def _dense3(a, b, hs2, dinv, b2):
    return pl.pallas_call(
        _dense3_body,
        grid=(NPAD // _R,),
        in_specs=[
            pl.BlockSpec((_R, D_OUT), lambda i: (i, 0)),
            pl.BlockSpec((_R, D_OUT), lambda i: (i, 0)),
            pl.BlockSpec((_R, D_OUT), lambda i: (i, 0)),
            pl.BlockSpec((_R, 1), lambda i: (i, 0)),
            pl.BlockSpec((1, D_OUT), lambda i: (0, 0)),
        ],
        out_specs=pl.BlockSpec((_R, D_OUT), lambda i: (i, 0)),
        out_shape=jax.ShapeDtypeStruct((NPAD, D_OUT), jnp.float32),
    )(a, b, hs2, dinv, b2)


def kernel(x, edge_index, W1, b1, gamma, beta, W2, b2, W3, b3):
    ei = jnp.pad(edge_index, ((0, 0), (0, EPAD - E)),
                 constant_values=N).reshape(2, 32, NCPT, CH)
    xp = jnp.pad(x, ((0, NPAD - N), (0, 0)))
    zpad = jnp.zeros((NPAD,), jnp.float32)
    degp = _deg_kernel(ei, zpad)
    dega = degp[0].reshape(NPAD, 1)
    degb = degp[1].reshape(NPAD, 1)
    hs1, dinv = _dense1(xp, W1, dega, degb)
    p1 = _agg_kernel(hs1, ei)
    hs2, out2 = _dense2(p1[0], p1[1], hs1, dinv,
                        b1.reshape(1, D_OUT), gamma.reshape(1, D_OUT),
                        beta.reshape(1, D_OUT), W2, W3, b3.reshape(1, 6))
    p2 = _agg_kernel(hs2, ei)
    out1 = _dense3(p2[0], p2[1], hs2, dinv, b2.reshape(1, D_OUT))
    return (out1[:N], out2[:N])


# speedup vs baseline: 2.4445x; 2.4445x over previous
import functools

import jax
import jax.numpy as jnp
from jax import lax
from jax.experimental import pallas as pl
from jax.experimental.pallas import tpu as pltpu
from jax.experimental.pallas import tpu_sc as plsc

N = 10000
NPAD = 10240
E = 320000
CH = 128
NCPT = 80
EPT = NCPT * CH
EPAD = 32 * EPT
NBUF = 4
D_IN = 128
D_OUT = 64
ROWS_PT = NPAD // 16

_mesh = plsc.VectorSubcoreMesh(core_axis_name="c", subcore_axis_name="s")


@functools.partial(
    pl.kernel,
    out_type=jax.ShapeDtypeStruct((2, NPAD), jnp.float32),
    mesh=_mesh,
    compiler_params=pltpu.CompilerParams(use_tc_tiling_on_sc=False),
    scratch_types=[
        pltpu.VMEM((NCPT, CH), jnp.int32),
        pltpu.VMEM((CH,), jnp.float32),
        pltpu.VMEM_SHARED((NPAD,), jnp.float32),
        pltpu.SemaphoreType.DMA,
    ],
)
def _deg_kernel(ei_hbm, zeros_hbm, out_hbm, idx_d, ones_v, acc, sem):
    cid = lax.axis_index("c")
    sid = lax.axis_index("s")
    wid = cid * 16 + sid
    for i in range(CH // 16):
        ones_v[pl.ds(i * 16, 16)] = jnp.full((16,), 1.0, jnp.float32)
    pltpu.sync_copy(ei_hbm.at[1, wid], idx_d)
    pltpu.sync_copy(zeros_hbm.at[pl.ds(sid * ROWS_PT, ROWS_PT)],
                    acc.at[pl.ds(sid * ROWS_PT, ROWS_PT)])
    plsc.subcore_barrier()

    def fire(c, carry):
        pltpu.async_copy(ones_v, acc.at[idx_d.at[c]], sem, add=True)
        return carry

    def drain(c, carry):
        pltpu.make_async_copy(zeros_hbm.at[pl.ds(0, CH)], ones_v, sem).wait()
        return carry

    def group(t, carry):
        lax.fori_loop(16 * t, 16 * t + 16, fire, 0)
        lax.fori_loop(0, 16, drain, 0)
        return carry

    lax.fori_loop(0, NCPT // 16, group, 0)
    plsc.subcore_barrier()
    pltpu.sync_copy(acc.at[pl.ds(sid * ROWS_PT, ROWS_PT)],
                    out_hbm.at[cid, pl.ds(sid * ROWS_PT, ROWS_PT)])


@functools.partial(
    pl.kernel,
    out_type=jax.ShapeDtypeStruct((2, NPAD, D_OUT), jnp.float32),
    mesh=_mesh,
    compiler_params=pltpu.CompilerParams(use_tc_tiling_on_sc=False),
    scratch_types=[
        pltpu.VMEM((NCPT, CH), jnp.int32),
        pltpu.VMEM((NCPT, CH), jnp.int32),
        pltpu.VMEM((NBUF, CH, D_OUT), jnp.float32),
        pltpu.VMEM_SHARED((NPAD, D_OUT), jnp.float32),
        [pltpu.SemaphoreType.DMA] * NBUF,
        [pltpu.SemaphoreType.DMA] * NBUF,
    ],
)
def _agg_kernel(hs_hbm, ei_hbm, out_hbm, idx_s, idx_d, rows, acc, gsem, ssem):
    cid = lax.axis_index("c")
    sid = lax.axis_index("s")
    wid = cid * 16 + sid
    pltpu.sync_copy(ei_hbm.at[0, wid], idx_s)
    pltpu.sync_copy(ei_hbm.at[1, wid], idx_d)
    pltpu.sync_copy(hs_hbm.at[pl.ds(sid * ROWS_PT, ROWS_PT)],
                    acc.at[pl.ds(sid * ROWS_PT, ROWS_PT)])
    plsc.subcore_barrier()

    def group(t, carry):
        for b in range(NBUF):
            c = NBUF * t + b

            @pl.when(t > 0)
            def _():
                pltpu.make_async_copy(hs_hbm.at[pl.ds(0, CH)], rows.at[b],
                                      ssem[b]).wait()

            pltpu.async_copy(hs_hbm.at[idx_s.at[c]], rows.at[b], gsem[b])
        for b in range(NBUF):
            c = NBUF * t + b
            pltpu.make_async_copy(hs_hbm.at[pl.ds(0, CH)], rows.at[b],
                                  gsem[b]).wait()
            pltpu.async_copy(rows.at[b], acc.at[idx_d.at[c]], ssem[b],
                             add=True)
        return carry

    lax.fori_loop(0, NCPT // NBUF, group, 0)
    for b in range(NBUF):
        pltpu.make_async_copy(hs_hbm.at[pl.ds(0, CH)], rows.at[b],
                              ssem[b]).wait()
    plsc.subcore_barrier()
    pltpu.sync_copy(acc.at[pl.ds(sid * ROWS_PT, ROWS_PT)],
                    out_hbm.at[cid, pl.ds(sid * ROWS_PT, ROWS_PT)])


_R = 640


def _dense1_body(x_ref, w_ref, dega_ref, degb_ref, hs_ref, dinv_ref):
    h = jnp.dot(x_ref[...], w_ref[...], preferred_element_type=jnp.float32)
    deg = dega_ref[...] + degb_ref[...] + 1.0
    dinv = lax.rsqrt(deg)
    hs_ref[...] = h * dinv
    dinv_ref[...] = dinv


def _dense1(x, W1, dega, degb):
    return pl.pallas_call(
        _dense1_body,
        grid=(NPAD // _R,),
        in_specs=[
            pl.BlockSpec((_R, D_IN), lambda i: (i, 0)),
            pl.BlockSpec((D_IN, D_OUT), lambda i: (0, 0)),
            pl.BlockSpec((_R, 1), lambda i: (i, 0)),
            pl.BlockSpec((_R, 1), lambda i: (i, 0)),
        ],
        out_specs=[
            pl.BlockSpec((_R, D_OUT), lambda i: (i, 0)),
            pl.BlockSpec((_R, 1), lambda i: (i, 0)),
        ],
        out_shape=[
            jax.ShapeDtypeStruct((NPAD, D_OUT), jnp.float32),
            jax.ShapeDtypeStruct((NPAD, 1), jnp.float32),
        ],
    )(x, W1, dega, degb)


def _dense2_body(a_ref, b_ref, hs1_ref, dinv_ref, b1_ref, g_ref, be_ref,
                 w2_ref, w3_ref, b3_ref, hs2_ref, out2_ref):
    dinv = dinv_ref[...]
    y = (a_ref[...] + b_ref[...] - hs1_ref[...]) * dinv + b1_ref[...]
    mu = jnp.mean(y, axis=-1, keepdims=True)
    d = y - mu
    var = jnp.mean(d * d, axis=-1, keepdims=True)
    hr = d * lax.rsqrt(var + 1e-5) * g_ref[...] + be_ref[...]
    hr = jnp.maximum(hr, 0.0)
    hs2_ref[...] = jnp.dot(hr, w2_ref[...],
                           preferred_element_type=jnp.float32) * dinv
    z = jnp.dot(hr, w3_ref[...], preferred_element_type=jnp.float32)
    out2_ref[...] = jax.nn.sigmoid(z + b3_ref[...])


def _dense2(a, b, hs1, dinv, b1, gamma, beta, W2, W3, b3):
    return pl.pallas_call(
        _dense2_body,
        grid=(NPAD // _R,),
        in_specs=[
            pl.BlockSpec((_R, D_OUT), lambda i: (i, 0)),
            pl.BlockSpec((_R, D_OUT), lambda i: (i, 0)),
            pl.BlockSpec((_R, D_OUT), lambda i: (i, 0)),
            pl.BlockSpec((_R, 1), lambda i: (i, 0)),
            pl.BlockSpec((1, D_OUT), lambda i: (0, 0)),
            pl.BlockSpec((1, D_OUT), lambda i: (0, 0)),
            pl.BlockSpec((1, D_OUT), lambda i: (0, 0)),
            pl.BlockSpec((D_OUT, D_OUT), lambda i: (0, 0)),
            pl.BlockSpec((D_OUT, 6), lambda i: (0, 0)),
            pl.BlockSpec((1, 6), lambda i: (0, 0)),
        ],
        out_specs=[
            pl.BlockSpec((_R, D_OUT), lambda i: (i, 0)),
            pl.BlockSpec((_R, 6), lambda i: (i, 0)),
        ],
        out_shape=[
            jax.ShapeDtypeStruct((NPAD, D_OUT), jnp.float32),
            jax.ShapeDtypeStruct((NPAD, 6), jnp.float32),
        ],
    )(a, b, hs1, dinv, b1, gamma, beta, W2, W3, b3)


def _dense3_body(a_ref, b_ref, hs2_ref, dinv_ref, b2_ref, out_ref):
    out_ref[...] = ((a_ref[...] + b_ref[...] - hs2_ref[...]) * dinv_ref[...]
                    + b2_ref[...])


def _dense3(a, b, hs2, dinv, b2):
    return pl.pallas_call(
        _dense3_body,
        grid=(NPAD // _R,),
        in_specs=[
            pl.BlockSpec((_R, D_OUT), lambda i: (i, 0)),
            pl.BlockSpec((_R, D_OUT), lambda i: (i, 0)),
            pl.BlockSpec((_R, D_OUT), lambda i: (i, 0)),
            pl.BlockSpec((_R, 1), lambda i: (i, 0)),
            pl.BlockSpec((1, D_OUT), lambda i: (0, 0)),
        ],
        out_specs=pl.BlockSpec((_R, D_OUT), lambda i: (i, 0)),
        out_shape=jax.ShapeDtypeStruct((NPAD, D_OUT), jnp.float32),
    )(a, b, hs2, dinv, b2)


def kernel(x, edge_index, W1, b1, gamma, beta, W2, b2, W3, b3):
    pad = (N + jnp.arange(EPAD - E, dtype=edge_index.dtype) % (NPAD - N))
    ei = jnp.concatenate(
        [edge_index, jnp.stack([pad, pad])], axis=1).reshape(2, 32, NCPT, CH)
    xp = jnp.pad(x, ((0, NPAD - N), (0, 0)))
    zpad = jnp.zeros((NPAD,), jnp.float32)
    degp = _deg_kernel(ei, zpad)
    dega = degp[0].reshape(NPAD, 1)
    degb = degp[1].reshape(NPAD, 1)
    hs1, dinv = _dense1(xp, W1, dega, degb)
    p1 = _agg_kernel(hs1, ei)
    hs2, out2 = _dense2(p1[0], p1[1], hs1, dinv,
                        b1.reshape(1, D_OUT), gamma.reshape(1, D_OUT),
                        beta.reshape(1, D_OUT), W2, W3, b3.reshape(1, 6))
    p2 = _agg_kernel(hs2, ei)
    out1 = _dense3(p2[0], p2[1], hs2, dinv, b2.reshape(1, D_OUT))
    return (out1[:N], out2[:N])


# speedup vs baseline: 2.5142x; 1.0285x over previous
import functools

import jax
import jax.numpy as jnp
from jax import lax
from jax.experimental import pallas as pl
from jax.experimental.pallas import tpu as pltpu
from jax.experimental.pallas import tpu_sc as plsc

N = 10000
NPAD = 10240
E = 320000
CH = 128
NCPT = 80
EPT = NCPT * CH
EPAD = 32 * EPT
NBUF = 8
D_IN = 128
D_OUT = 64
ROWS_PT = NPAD // 16

_mesh = plsc.VectorSubcoreMesh(core_axis_name="c", subcore_axis_name="s")


@functools.partial(
    pl.kernel,
    out_type=jax.ShapeDtypeStruct((2, NPAD), jnp.float32),
    mesh=_mesh,
    compiler_params=pltpu.CompilerParams(use_tc_tiling_on_sc=False),
    scratch_types=[
        pltpu.VMEM((NCPT, CH), jnp.int32),
        pltpu.VMEM((CH,), jnp.float32),
        pltpu.VMEM_SHARED((NPAD,), jnp.float32),
        pltpu.SemaphoreType.DMA,
    ],
)
def _deg_kernel(ei_hbm, zeros_hbm, out_hbm, idx_d, ones_v, acc, sem):
    cid = lax.axis_index("c")
    sid = lax.axis_index("s")
    wid = cid * 16 + sid
    for i in range(CH // 16):
        ones_v[pl.ds(i * 16, 16)] = jnp.full((16,), 1.0, jnp.float32)
    pltpu.sync_copy(ei_hbm.at[1, wid], idx_d)
    pltpu.sync_copy(zeros_hbm.at[pl.ds(sid * ROWS_PT, ROWS_PT)],
                    acc.at[pl.ds(sid * ROWS_PT, ROWS_PT)])
    plsc.subcore_barrier()

    def fire(c, carry):
        pltpu.async_copy(ones_v, acc.at[idx_d.at[c]], sem, add=True)
        return carry

    def drain(c, carry):
        pltpu.make_async_copy(zeros_hbm.at[pl.ds(0, CH)], ones_v, sem).wait()
        return carry

    def group(t, carry):
        lax.fori_loop(16 * t, 16 * t + 16, fire, 0)
        lax.fori_loop(0, 16, drain, 0)
        return carry

    lax.fori_loop(0, NCPT // 16, group, 0)
    plsc.subcore_barrier()
    pltpu.sync_copy(acc.at[pl.ds(sid * ROWS_PT, ROWS_PT)],
                    out_hbm.at[cid, pl.ds(sid * ROWS_PT, ROWS_PT)])


@functools.partial(
    pl.kernel,
    out_type=jax.ShapeDtypeStruct((2, NPAD, D_OUT), jnp.float32),
    mesh=_mesh,
    compiler_params=pltpu.CompilerParams(use_tc_tiling_on_sc=False),
    scratch_types=[
        pltpu.VMEM((NCPT, CH), jnp.int32),
        pltpu.VMEM((NCPT, CH), jnp.int32),
        pltpu.VMEM((NBUF, CH, D_OUT), jnp.float32),
        pltpu.VMEM_SHARED((NPAD, D_OUT), jnp.float32),
        [pltpu.SemaphoreType.DMA] * NBUF,
        [pltpu.SemaphoreType.DMA] * NBUF,
    ],
)
def _agg_kernel(hs_hbm, ei_hbm, out_hbm, idx_s, idx_d, rows, acc, gsem, ssem):
    cid = lax.axis_index("c")
    sid = lax.axis_index("s")
    wid = cid * 16 + sid
    pltpu.sync_copy(ei_hbm.at[0, wid], idx_s)
    pltpu.sync_copy(ei_hbm.at[1, wid], idx_d)
    pltpu.sync_copy(hs_hbm.at[pl.ds(sid * ROWS_PT, ROWS_PT)],
                    acc.at[pl.ds(sid * ROWS_PT, ROWS_PT)])
    plsc.subcore_barrier()

    def group(t, carry):
        for b in range(NBUF):
            c = NBUF * t + b

            @pl.when(t > 0)
            def _():
                pltpu.make_async_copy(hs_hbm.at[pl.ds(0, CH)], rows.at[b],
                                      ssem[b]).wait()

            pltpu.async_copy(hs_hbm.at[idx_s.at[c]], rows.at[b], gsem[b])
        for b in range(NBUF):
            c = NBUF * t + b
            pltpu.make_async_copy(hs_hbm.at[pl.ds(0, CH)], rows.at[b],
                                  gsem[b]).wait()
            pltpu.async_copy(rows.at[b], acc.at[idx_d.at[c]], ssem[b],
                             add=True)
        return carry

    lax.fori_loop(0, NCPT // NBUF, group, 0)
    for b in range(NBUF):
        pltpu.make_async_copy(hs_hbm.at[pl.ds(0, CH)], rows.at[b],
                              ssem[b]).wait()
    plsc.subcore_barrier()
    pltpu.sync_copy(acc.at[pl.ds(sid * ROWS_PT, ROWS_PT)],
                    out_hbm.at[cid, pl.ds(sid * ROWS_PT, ROWS_PT)])


_R = 640


def _dense1_body(x_ref, w_ref, dega_ref, degb_ref, hs_ref, dinv_ref):
    h = jnp.dot(x_ref[...], w_ref[...], preferred_element_type=jnp.float32)
    deg = dega_ref[...] + degb_ref[...] + 1.0
    dinv = lax.rsqrt(deg)
    hs_ref[...] = h * dinv
    dinv_ref[...] = dinv


def _dense1(x, W1, dega, degb):
    return pl.pallas_call(
        _dense1_body,
        grid=(NPAD // _R,),
        in_specs=[
            pl.BlockSpec((_R, D_IN), lambda i: (i, 0)),
            pl.BlockSpec((D_IN, D_OUT), lambda i: (0, 0)),
            pl.BlockSpec((_R, 1), lambda i: (i, 0)),
            pl.BlockSpec((_R, 1), lambda i: (i, 0)),
        ],
        out_specs=[
            pl.BlockSpec((_R, D_OUT), lambda i: (i, 0)),
            pl.BlockSpec((_R, 1), lambda i: (i, 0)),
        ],
        out_shape=[
            jax.ShapeDtypeStruct((NPAD, D_OUT), jnp.float32),
            jax.ShapeDtypeStruct((NPAD, 1), jnp.float32),
        ],
    )(x, W1, dega, degb)


def _dense2_body(a_ref, b_ref, hs1_ref, dinv_ref, b1_ref, g_ref, be_ref,
                 w2_ref, w3_ref, b3_ref, hs2_ref, out2_ref):
    dinv = dinv_ref[...]
    y = (a_ref[...] + b_ref[...] - hs1_ref[...]) * dinv + b1_ref[...]
    mu = jnp.mean(y, axis=-1, keepdims=True)
    d = y - mu
    var = jnp.mean(d * d, axis=-1, keepdims=True)
    hr = d * lax.rsqrt(var + 1e-5) * g_ref[...] + be_ref[...]
    hr = jnp.maximum(hr, 0.0)
    hs2_ref[...] = jnp.dot(hr, w2_ref[...],
                           preferred_element_type=jnp.float32) * dinv
    z = jnp.dot(hr, w3_ref[...], preferred_element_type=jnp.float32)
    out2_ref[...] = jax.nn.sigmoid(z + b3_ref[...])


def _dense2(a, b, hs1, dinv, b1, gamma, beta, W2, W3, b3):
    return pl.pallas_call(
        _dense2_body,
        grid=(NPAD // _R,),
        in_specs=[
            pl.BlockSpec((_R, D_OUT), lambda i: (i, 0)),
            pl.BlockSpec((_R, D_OUT), lambda i: (i, 0)),
            pl.BlockSpec((_R, D_OUT), lambda i: (i, 0)),
            pl.BlockSpec((_R, 1), lambda i: (i, 0)),
            pl.BlockSpec((1, D_OUT), lambda i: (0, 0)),
            pl.BlockSpec((1, D_OUT), lambda i: (0, 0)),
            pl.BlockSpec((1, D_OUT), lambda i: (0, 0)),
            pl.BlockSpec((D_OUT, D_OUT), lambda i: (0, 0)),
            pl.BlockSpec((D_OUT, 6), lambda i: (0, 0)),
            pl.BlockSpec((1, 6), lambda i: (0, 0)),
        ],
        out_specs=[
            pl.BlockSpec((_R, D_OUT), lambda i: (i, 0)),
            pl.BlockSpec((_R, 6), lambda i: (i, 0)),
        ],
        out_shape=[
            jax.ShapeDtypeStruct((NPAD, D_OUT), jnp.float32),
            jax.ShapeDtypeStruct((NPAD, 6), jnp.float32),
        ],
    )(a, b, hs1, dinv, b1, gamma, beta, W2, W3, b3)


def _dense3_body(a_ref, b_ref, hs2_ref, dinv_ref, b2_ref, out_ref):
    out_ref[...] = ((a_ref[...] + b_ref[...] - hs2_ref[...]) * dinv_ref[...]
                    + b2_ref[...])


def _dense3(a, b, hs2, dinv, b2):
    return pl.pallas_call(
        _dense3_body,
        grid=(NPAD // _R,),
        in_specs=[
            pl.BlockSpec((_R, D_OUT), lambda i: (i, 0)),
            pl.BlockSpec((_R, D_OUT), lambda i: (i, 0)),
            pl.BlockSpec((_R, D_OUT), lambda i: (i, 0)),
            pl.BlockSpec((_R, 1), lambda i: (i, 0)),
            pl.BlockSpec((1, D_OUT), lambda i: (0, 0)),
        ],
        out_specs=pl.BlockSpec((_R, D_OUT), lambda i: (i, 0)),
        out_shape=jax.ShapeDtypeStruct((NPAD, D_OUT), jnp.float32),
    )(a, b, hs2, dinv, b2)


def kernel(x, edge_index, W1, b1, gamma, beta, W2, b2, W3, b3):
    pad = (N + jnp.arange(EPAD - E, dtype=edge_index.dtype) % (NPAD - N))
    ei = jnp.concatenate(
        [edge_index, jnp.stack([pad, pad])], axis=1).reshape(2, 32, NCPT, CH)
    xp = jnp.pad(x, ((0, NPAD - N), (0, 0)))
    zpad = jnp.zeros((NPAD,), jnp.float32)
    degp = _deg_kernel(ei, zpad)
    dega = degp[0].reshape(NPAD, 1)
    degb = degp[1].reshape(NPAD, 1)
    hs1, dinv = _dense1(xp, W1, dega, degb)
    p1 = _agg_kernel(hs1, ei)
    hs2, out2 = _dense2(p1[0], p1[1], hs1, dinv,
                        b1.reshape(1, D_OUT), gamma.reshape(1, D_OUT),
                        beta.reshape(1, D_OUT), W2, W3, b3.reshape(1, 6))
    p2 = _agg_kernel(hs2, ei)
    out1 = _dense3(p2[0], p2[1], hs2, dinv, b2.reshape(1, D_OUT))
    return (out1[:N], out2[:N])


# speedup vs baseline: 2.7062x; 1.0764x over previous
import functools

import jax
import jax.numpy as jnp
from jax import lax
from jax.experimental import pallas as pl
from jax.experimental.pallas import tpu as pltpu
from jax.experimental.pallas import tpu_sc as plsc

N = 10000
NPAD = 10240
E = 320000
CH = 128
NCPT = 80
EPT = NCPT * CH
EPAD = 32 * EPT
NBUF = 8
D_IN = 128
D_OUT = 64
ROWS_PT = NPAD // 16

_mesh = plsc.VectorSubcoreMesh(core_axis_name="c", subcore_axis_name="s")


@functools.partial(
    pl.kernel,
    out_type=jax.ShapeDtypeStruct((2, NPAD), jnp.float32),
    mesh=_mesh,
    compiler_params=pltpu.CompilerParams(use_tc_tiling_on_sc=False),
    scratch_types=[
        pltpu.VMEM((NCPT, CH), jnp.int32),
        pltpu.VMEM((CH,), jnp.float32),
        pltpu.VMEM_SHARED((NPAD,), jnp.float32),
        pltpu.SemaphoreType.DMA,
    ],
)
def _deg_kernel(ei_hbm, zeros_hbm, out_hbm, idx_d, ones_v, acc, sem):
    cid = lax.axis_index("c")
    sid = lax.axis_index("s")
    wid = cid * 16 + sid
    for i in range(CH // 16):
        ones_v[pl.ds(i * 16, 16)] = jnp.full((16,), 1.0, jnp.float32)
    pltpu.sync_copy(ei_hbm.at[1, wid], idx_d)
    pltpu.sync_copy(zeros_hbm.at[pl.ds(sid * ROWS_PT, ROWS_PT)],
                    acc.at[pl.ds(sid * ROWS_PT, ROWS_PT)])
    plsc.subcore_barrier()

    def fire(c, carry):
        pltpu.async_copy(ones_v, acc.at[idx_d.at[c]], sem, add=True)
        return carry

    def drain(c, carry):
        pltpu.make_async_copy(zeros_hbm.at[pl.ds(0, CH)], ones_v, sem).wait()
        return carry

    def group(t, carry):
        lax.fori_loop(16 * t, 16 * t + 16, fire, 0)
        lax.fori_loop(0, 16, drain, 0)
        return carry

    lax.fori_loop(0, NCPT // 16, group, 0)
    plsc.subcore_barrier()
    pltpu.sync_copy(acc.at[pl.ds(sid * ROWS_PT, ROWS_PT)],
                    out_hbm.at[cid, pl.ds(sid * ROWS_PT, ROWS_PT)])


@functools.partial(
    pl.kernel,
    out_type=jax.ShapeDtypeStruct((2, NPAD, D_OUT), jnp.float32),
    mesh=_mesh,
    compiler_params=pltpu.CompilerParams(use_tc_tiling_on_sc=False),
    scratch_types=[
        pltpu.VMEM((NCPT, CH), jnp.int32),
        pltpu.VMEM((NCPT, CH), jnp.int32),
        pltpu.VMEM((NBUF, CH, D_OUT), jnp.float32),
        pltpu.VMEM_SHARED((NPAD, D_OUT), jnp.float32),
        [pltpu.SemaphoreType.DMA] * NBUF,
        [pltpu.SemaphoreType.DMA] * NBUF,
    ],
)
def _agg_kernel(hs_hbm, ei_hbm, out_hbm, idx_s, idx_d, rows, acc, gsem, ssem):
    cid = lax.axis_index("c")
    sid = lax.axis_index("s")
    wid = cid * 16 + sid
    pltpu.sync_copy(ei_hbm.at[0, wid], idx_s)
    pltpu.sync_copy(ei_hbm.at[1, wid], idx_d)
    pltpu.sync_copy(hs_hbm.at[pl.ds(sid * ROWS_PT, ROWS_PT)],
                    acc.at[pl.ds(sid * ROWS_PT, ROWS_PT)])
    plsc.subcore_barrier()

    def group(t, carry):
        for b in range(NBUF):
            c = NBUF * t + b

            @pl.when(t > 0)
            def _():
                pltpu.make_async_copy(hs_hbm.at[pl.ds(0, CH)], rows.at[b],
                                      ssem[b]).wait()

            pltpu.async_copy(hs_hbm.at[idx_s.at[c]], rows.at[b], gsem[b])
        for b in range(NBUF):
            c = NBUF * t + b
            pltpu.make_async_copy(hs_hbm.at[pl.ds(0, CH)], rows.at[b],
                                  gsem[b]).wait()
            pltpu.async_copy(rows.at[b], acc.at[idx_d.at[c]], ssem[b],
                             add=True)
        return carry

    lax.fori_loop(0, NCPT // NBUF, group, 0)
    for b in range(NBUF):
        pltpu.make_async_copy(hs_hbm.at[pl.ds(0, CH)], rows.at[b],
                              ssem[b]).wait()
    plsc.subcore_barrier()
    pltpu.sync_copy(acc.at[pl.ds(sid * ROWS_PT, ROWS_PT)],
                    out_hbm.at[cid, pl.ds(sid * ROWS_PT, ROWS_PT)])


_R = 640


def _dense1_body(x_ref, w_ref, dega_ref, degb_ref, hs_ref, dinv_ref):
    h = jnp.dot(x_ref[...], w_ref[...], preferred_element_type=jnp.float32)
    deg = dega_ref[...] + degb_ref[...] + 1.0
    dinv = lax.rsqrt(deg)
    hs_ref[...] = h * dinv
    dinv_ref[...] = dinv


def _dense1(x, W1, degf):
    return pl.pallas_call(
        _dense1_body,
        grid=(NPAD // _R,),
        in_specs=[
            pl.BlockSpec((_R, D_IN), lambda i: (i, 0)),
            pl.BlockSpec((D_IN, D_OUT), lambda i: (0, 0)),
            pl.BlockSpec((_R, 1), lambda i: (i, 0)),
            pl.BlockSpec((_R, 1), lambda i: (i + NPAD // _R, 0)),
        ],
        out_specs=[
            pl.BlockSpec((_R, D_OUT), lambda i: (i, 0)),
            pl.BlockSpec((_R, 1), lambda i: (i, 0)),
        ],
        out_shape=[
            jax.ShapeDtypeStruct((NPAD, D_OUT), jnp.float32),
            jax.ShapeDtypeStruct((NPAD, 1), jnp.float32),
        ],
    )(x, W1, degf, degf)


def _dense2_body(a_ref, b_ref, hs1_ref, dinv_ref, b1_ref, g_ref, be_ref,
                 w2_ref, w3_ref, b3_ref, hs2_ref, out2_ref):
    dinv = dinv_ref[...]
    y = (a_ref[0] + b_ref[0] - hs1_ref[...]) * dinv + b1_ref[...]
    mu = jnp.mean(y, axis=-1, keepdims=True)
    d = y - mu
    var = jnp.mean(d * d, axis=-1, keepdims=True)
    hr = d * lax.rsqrt(var + 1e-5) * g_ref[...] + be_ref[...]
    hr = jnp.maximum(hr, 0.0)
    hs2_ref[...] = jnp.dot(hr, w2_ref[...],
                           preferred_element_type=jnp.float32) * dinv
    z = jnp.dot(hr, w3_ref[...], preferred_element_type=jnp.float32)
    out2_ref[...] = jax.nn.sigmoid(z + b3_ref[...])


def _dense2(p, hs1, dinv, b1, gamma, beta, W2, W3, b3):
    return pl.pallas_call(
        _dense2_body,
        grid=(NPAD // _R,),
        in_specs=[
            pl.BlockSpec((1, _R, D_OUT), lambda i: (0, i, 0)),
            pl.BlockSpec((1, _R, D_OUT), lambda i: (1, i, 0)),
            pl.BlockSpec((_R, D_OUT), lambda i: (i, 0)),
            pl.BlockSpec((_R, 1), lambda i: (i, 0)),
            pl.BlockSpec((1, D_OUT), lambda i: (0, 0)),
            pl.BlockSpec((1, D_OUT), lambda i: (0, 0)),
            pl.BlockSpec((1, D_OUT), lambda i: (0, 0)),
            pl.BlockSpec((D_OUT, D_OUT), lambda i: (0, 0)),
            pl.BlockSpec((D_OUT, 6), lambda i: (0, 0)),
            pl.BlockSpec((1, 6), lambda i: (0, 0)),
        ],
        out_specs=[
            pl.BlockSpec((_R, D_OUT), lambda i: (i, 0)),
            pl.BlockSpec((_R, 6), lambda i: (i, 0)),
        ],
        out_shape=[
            jax.ShapeDtypeStruct((NPAD, D_OUT), jnp.float32),
            jax.ShapeDtypeStruct((NPAD, 6), jnp.float32),
        ],
    )(p, p, hs1, dinv, b1, gamma, beta, W2, W3, b3)


def _dense3_body(a_ref, b_ref, hs2_ref, dinv_ref, b2_ref, out_ref):
    out_ref[...] = ((a_ref[0] + b_ref[0] - hs2_ref[...]) * dinv_ref[...]
                    + b2_ref[...])


_R3 = 2000


def _dense3(p, hs2, dinv, b2):
    return pl.pallas_call(
        _dense3_body,
        grid=(N // _R3,),
        in_specs=[
            pl.BlockSpec((1, _R3, D_OUT), lambda i: (0, i, 0)),
            pl.BlockSpec((1, _R3, D_OUT), lambda i: (1, i, 0)),
            pl.BlockSpec((_R3, D_OUT), lambda i: (i, 0)),
            pl.BlockSpec((_R3, 1), lambda i: (i, 0)),
            pl.BlockSpec((1, D_OUT), lambda i: (0, 0)),
        ],
        out_specs=pl.BlockSpec((_R3, D_OUT), lambda i: (i, 0)),
        out_shape=jax.ShapeDtypeStruct((N, D_OUT), jnp.float32),
    )(p, p, hs2, dinv, b2)


def kernel(x, edge_index, W1, b1, gamma, beta, W2, b2, W3, b3):
    pad = (N + jnp.arange(EPAD - E, dtype=edge_index.dtype) % (NPAD - N))
    ei = jnp.concatenate(
        [edge_index, jnp.stack([pad, pad])], axis=1).reshape(2, 32, NCPT, CH)
    xp = jnp.pad(x, ((0, NPAD - N), (0, 0)))
    zpad = jnp.zeros((NPAD,), jnp.float32)
    degp = _deg_kernel(ei, zpad)
    degf = degp.reshape(2 * NPAD, 1)
    hs1, dinv = _dense1(xp, W1, degf)
    p1 = _agg_kernel(hs1, ei)
    hs2, out2 = _dense2(p1, hs1, dinv,
                        b1.reshape(1, D_OUT), gamma.reshape(1, D_OUT),
                        beta.reshape(1, D_OUT), W2, W3, b3.reshape(1, 6))
    p2 = _agg_kernel(hs2, ei)
    out1 = _dense3(p2, hs2, dinv, b2.reshape(1, D_OUT))
    return (out1, out2[:N])


# speedup vs baseline: 2.7242x; 1.0067x over previous
import functools

import jax
import jax.numpy as jnp
from jax import lax
from jax.experimental import pallas as pl
from jax.experimental.pallas import tpu as pltpu
from jax.experimental.pallas import tpu_sc as plsc

N = 10000
NPAD = 10240
E = 320000
CH = 128
NCPT = 80
EPT = NCPT * CH
EPAD = 32 * EPT
NBUF = 8
D_IN = 128
D_OUT = 64
ROWS_PT = NPAD // 16

_mesh = plsc.VectorSubcoreMesh(core_axis_name="c", subcore_axis_name="s")


@functools.partial(
    pl.kernel,
    out_type=jax.ShapeDtypeStruct((2, NPAD), jnp.float32),
    mesh=_mesh,
    compiler_params=pltpu.CompilerParams(use_tc_tiling_on_sc=False),
    scratch_types=[
        pltpu.VMEM((NCPT, CH), jnp.int32),
        pltpu.VMEM((CH,), jnp.float32),
        pltpu.VMEM_SHARED((NPAD,), jnp.float32),
        pltpu.SemaphoreType.DMA,
    ],
)
def _deg_kernel(ei_hbm, zeros_hbm, out_hbm, idx_d, ones_v, acc, sem):
    cid = lax.axis_index("c")
    sid = lax.axis_index("s")
    wid = cid * 16 + sid
    for i in range(CH // 16):
        ones_v[pl.ds(i * 16, 16)] = jnp.full((16,), 1.0, jnp.float32)
    pltpu.sync_copy(ei_hbm.at[1, wid], idx_d)
    pltpu.sync_copy(zeros_hbm.at[pl.ds(sid * ROWS_PT, ROWS_PT)],
                    acc.at[pl.ds(sid * ROWS_PT, ROWS_PT)])
    plsc.subcore_barrier()

    def fire(c, carry):
        pltpu.async_copy(ones_v, acc.at[idx_d.at[c]], sem, add=True)
        return carry

    def drain(c, carry):
        pltpu.make_async_copy(zeros_hbm.at[pl.ds(0, CH)], ones_v, sem).wait()
        return carry

    def group(t, carry):
        lax.fori_loop(16 * t, 16 * t + 16, fire, 0)
        lax.fori_loop(0, 16, drain, 0)
        return carry

    lax.fori_loop(0, NCPT // 16, group, 0)
    plsc.subcore_barrier()
    pltpu.sync_copy(acc.at[pl.ds(sid * ROWS_PT, ROWS_PT)],
                    out_hbm.at[cid, pl.ds(sid * ROWS_PT, ROWS_PT)])


@functools.partial(
    pl.kernel,
    out_type=jax.ShapeDtypeStruct((2, NPAD, D_OUT), jnp.float32),
    mesh=_mesh,
    compiler_params=pltpu.CompilerParams(use_tc_tiling_on_sc=False),
    scratch_types=[
        pltpu.VMEM((NCPT, CH), jnp.int32),
        pltpu.VMEM((NCPT, CH), jnp.int32),
        pltpu.VMEM((NBUF, CH, D_OUT), jnp.float32),
        pltpu.VMEM_SHARED((NPAD, D_OUT), jnp.float32),
        [pltpu.SemaphoreType.DMA] * NBUF,
        [pltpu.SemaphoreType.DMA] * NBUF,
    ],
)
def _agg_kernel(hs_hbm, ei_hbm, out_hbm, idx_s, idx_d, rows, acc, gsem, ssem):
    cid = lax.axis_index("c")
    sid = lax.axis_index("s")
    wid = cid * 16 + sid
    pltpu.sync_copy(ei_hbm.at[0, wid], idx_s)
    pltpu.sync_copy(ei_hbm.at[1, wid], idx_d)
    pltpu.sync_copy(hs_hbm.at[pl.ds(sid * ROWS_PT, ROWS_PT)],
                    acc.at[pl.ds(sid * ROWS_PT, ROWS_PT)])
    plsc.subcore_barrier()

    def group(t, carry):
        for b in range(NBUF):
            c = NBUF * t + b

            @pl.when(t > 0)
            def _():
                pltpu.make_async_copy(hs_hbm.at[pl.ds(0, CH)], rows.at[b],
                                      ssem[b]).wait()

            pltpu.async_copy(hs_hbm.at[idx_s.at[c]], rows.at[b], gsem[b])
        for b in range(NBUF):
            c = NBUF * t + b
            pltpu.make_async_copy(hs_hbm.at[pl.ds(0, CH)], rows.at[b],
                                  gsem[b]).wait()
            pltpu.async_copy(rows.at[b], acc.at[idx_d.at[c]], ssem[b],
                             add=True)
        return carry

    lax.fori_loop(0, NCPT // NBUF, group, 0)
    for b in range(NBUF):
        pltpu.make_async_copy(hs_hbm.at[pl.ds(0, CH)], rows.at[b],
                              ssem[b]).wait()
    plsc.subcore_barrier()
    pltpu.sync_copy(acc.at[pl.ds(sid * ROWS_PT, ROWS_PT)],
                    out_hbm.at[cid, pl.ds(sid * ROWS_PT, ROWS_PT)])


_R = 640


def _dense1_body(x_ref, w_ref, dega_ref, degb_ref, hs_ref, dinv_ref):
    h = jnp.dot(x_ref[...], w_ref[...], preferred_element_type=jnp.float32)
    deg = dega_ref[...] + degb_ref[...] + 1.0
    dinv = lax.rsqrt(deg)
    hs_ref[...] = h * dinv
    dinv_ref[...] = dinv


def _dense1(x, W1, degf):
    return pl.pallas_call(
        _dense1_body,
        grid=(NPAD // _R,),
        in_specs=[
            pl.BlockSpec((_R, D_IN), lambda i: (i, 0)),
            pl.BlockSpec((D_IN, D_OUT), lambda i: (0, 0)),
            pl.BlockSpec((_R, 1), lambda i: (i, 0)),
            pl.BlockSpec((_R, 1), lambda i: (i + NPAD // _R, 0)),
        ],
        out_specs=[
            pl.BlockSpec((_R, D_OUT), lambda i: (i, 0)),
            pl.BlockSpec((_R, 1), lambda i: (i, 0)),
        ],
        out_shape=[
            jax.ShapeDtypeStruct((NPAD, D_OUT), jnp.float32),
            jax.ShapeDtypeStruct((NPAD, 1), jnp.float32),
        ],
    )(x, W1, degf, degf)


def _dense2_body(a_ref, b_ref, hs1_ref, dinv_ref, b1_ref, g_ref, be_ref,
                 w2_ref, w3_ref, b3_ref, hs2_ref, out2_ref):
    dinv = dinv_ref[...]
    y = (a_ref[0] + b_ref[0] - hs1_ref[...]) * dinv + b1_ref[...]
    mu = jnp.mean(y, axis=-1, keepdims=True)
    d = y - mu
    var = jnp.mean(d * d, axis=-1, keepdims=True)
    hr = d * lax.rsqrt(var + 1e-5) * g_ref[...] + be_ref[...]
    hr = jnp.maximum(hr, 0.0)
    hs2_ref[...] = jnp.dot(hr, w2_ref[...],
                           preferred_element_type=jnp.float32) * dinv
    z = jnp.dot(hr, w3_ref[...], preferred_element_type=jnp.float32)
    out2_ref[...] = jax.nn.sigmoid(z + b3_ref[...])


def _dense2(p, hs1, dinv, b1, gamma, beta, W2, W3, b3):
    return pl.pallas_call(
        _dense2_body,
        grid=(NPAD // _R,),
        in_specs=[
            pl.BlockSpec((1, _R, D_OUT), lambda i: (0, i, 0)),
            pl.BlockSpec((1, _R, D_OUT), lambda i: (1, i, 0)),
            pl.BlockSpec((_R, D_OUT), lambda i: (i, 0)),
            pl.BlockSpec((_R, 1), lambda i: (i, 0)),
            pl.BlockSpec((1, D_OUT), lambda i: (0, 0)),
            pl.BlockSpec((1, D_OUT), lambda i: (0, 0)),
            pl.BlockSpec((1, D_OUT), lambda i: (0, 0)),
            pl.BlockSpec((D_OUT, D_OUT), lambda i: (0, 0)),
            pl.BlockSpec((D_OUT, 6), lambda i: (0, 0)),
            pl.BlockSpec((1, 6), lambda i: (0, 0)),
        ],
        out_specs=[
            pl.BlockSpec((_R, D_OUT), lambda i: (i, 0)),
            pl.BlockSpec((_R, 6), lambda i: (i, 0)),
        ],
        out_shape=[
            jax.ShapeDtypeStruct((NPAD, D_OUT), jnp.float32),
            jax.ShapeDtypeStruct((NPAD, 6), jnp.float32),
        ],
    )(p, p, hs1, dinv, b1, gamma, beta, W2, W3, b3)


def _dense3_body(a_ref, b_ref, hs2_ref, dinv_ref, b2_ref, out_ref):
    out_ref[...] = ((a_ref[0] + b_ref[0] - hs2_ref[...]) * dinv_ref[...]
                    + b2_ref[...])


_R3 = 2000


def _dense3(p, hs2, dinv, b2):
    return pl.pallas_call(
        _dense3_body,
        grid=(N // _R3,),
        in_specs=[
            pl.BlockSpec((1, _R3, D_OUT), lambda i: (0, i, 0)),
            pl.BlockSpec((1, _R3, D_OUT), lambda i: (1, i, 0)),
            pl.BlockSpec((_R3, D_OUT), lambda i: (i, 0)),
            pl.BlockSpec((_R3, 1), lambda i: (i, 0)),
            pl.BlockSpec((1, D_OUT), lambda i: (0, 0)),
        ],
        out_specs=pl.BlockSpec((_R3, D_OUT), lambda i: (i, 0)),
        out_shape=jax.ShapeDtypeStruct((N, D_OUT), jnp.float32),
    )(p, p, hs2, dinv, b2)


def kernel(x, edge_index, W1, b1, gamma, beta, W2, b2, W3, b3):
    pad = (N + jnp.arange(EPAD - E, dtype=edge_index.dtype) % (NPAD - N))
    ei = jnp.concatenate(
        [edge_index, jnp.stack([pad, pad])], axis=1).reshape(2, 32, NCPT, CH)
    zpad = jnp.zeros((NPAD,), jnp.float32)
    degp = _deg_kernel(ei, zpad)
    degf = degp.reshape(2 * NPAD, 1)
    hs1, dinv = _dense1(x, W1, degf)
    p1 = _agg_kernel(hs1, ei)
    hs2, out2 = _dense2(p1, hs1, dinv,
                        b1.reshape(1, D_OUT), gamma.reshape(1, D_OUT),
                        beta.reshape(1, D_OUT), W2, W3, b3.reshape(1, 6))
    p2 = _agg_kernel(hs2, ei)
    out1 = _dense3(p2, hs2, dinv, b2.reshape(1, D_OUT))
    return (out1, out2[:N])
